# baseline (speedup 1.0000x reference)
import functools

import jax
import jax.numpy as jnp
from jax.experimental import pallas as pl
from jax.experimental.pallas import tpu as pltpu

_ALPHA = 1.5
_DEPTH = 6
_NUM_LEAVES = 64
_TOP_K = 2
_N_ITER = 30


def _entmax_prep_body(fs_ref, sel_ref):
    x = fs_ref[...] * (_ALPHA - 1.0)
    max_val = jnp.max(x, axis=-1, keepdims=True)
    tau_lo = max_val - 1.0
    tau_hi = max_val

    def p_fn(tau):
        c = jnp.maximum(x - tau, 0.0)
        return c * c

    f_lo = jnp.sum(p_fn(tau_lo), axis=-1, keepdims=True) - 1.0
    for _ in range(_N_ITER):
        tau_m = 0.5 * (tau_lo + tau_hi)
        f_m = jnp.sum(p_fn(tau_m), axis=-1, keepdims=True) - 1.0
        same_sign = (f_m * f_lo) >= 0.0
        tau_lo = jnp.where(same_sign, tau_m, tau_lo)
        f_lo = jnp.where(same_sign, f_m, f_lo)
        tau_hi = jnp.where(same_sign, tau_hi, tau_m)
    p = p_fn(0.5 * (tau_lo + tau_hi))
    sel_ref[...] = p / jnp.sum(p, axis=-1, keepdims=True)


def _router_body(x_ref, selt_ref, thr_ref, lt_ref, w_ref_, b_ref,
                 logits_ref, idx_ref, wout_ref):
    t = x_ref.shape[0]
    fc = jnp.dot(x_ref[...], selt_ref[...], preferred_element_type=jnp.float32)
    fct = fc.T
    temp = jnp.exp(lt_ref[...])
    scaled = (fct - thr_ref[...]) / temp
    u = scaled * (_ALPHA - 1.0)
    max_val = jnp.maximum(u, 0.0)
    tau_lo = max_val - 1.0
    tau_hi = max_val

    def p2_fn(tau):
        c0 = jnp.maximum(u - tau, 0.0)
        c1 = jnp.maximum(-tau, 0.0)
        return c0 * c0, c1 * c1

    p0, p1 = p2_fn(tau_lo)
    f_lo = (p0 + p1) - 1.0
    for _ in range(_N_ITER):
        tau_m = 0.5 * (tau_lo + tau_hi)
        p0, p1 = p2_fn(tau_m)
        f_m = (p0 + p1) - 1.0
        same_sign = (f_m * f_lo) >= 0.0
        tau_lo = jnp.where(same_sign, tau_m, tau_lo)
        f_lo = jnp.where(same_sign, f_m, f_lo)
        tau_hi = jnp.where(same_sign, tau_hi, tau_m)
    p0, p1 = p2_fn(0.5 * (tau_lo + tau_hi))
    right = p0 / (p0 + p1)
    left = 1.0 - right
    bits_i = jax.lax.broadcasted_iota(jnp.int32, (_NUM_LEAVES, 1), 0)
    leaf = jnp.ones((_NUM_LEAVES, t), dtype=jnp.float32)
    for i in range(_DEPTH):
        bit_col = ((bits_i >> (_DEPTH - 1 - i)) & 1) == 1
        r_i = right[i:i + 1, :]
        l_i = left[i:i + 1, :]
        leaf = leaf * jnp.where(bit_col, r_i, l_i)
    logits = jnp.dot(w_ref_[...], leaf, preferred_element_type=jnp.float32)
    logits = logits + b_ref[...]
    logits_ref[...] = logits.T
    iota_f = jax.lax.broadcasted_iota(jnp.int32, (_NUM_LEAVES, t), 0).astype(
        jnp.float32)
    m1 = jnp.max(logits, axis=0, keepdims=True)
    a1 = jnp.min(jnp.where(logits == m1, iota_f, 64.0), axis=0, keepdims=True)
    masked = jnp.where(iota_f == a1, -jnp.inf, logits)
    m2 = jnp.max(masked, axis=0, keepdims=True)
    a2 = jnp.min(jnp.where(masked == m2, iota_f, 64.0), axis=0, keepdims=True)
    e = jnp.exp(m2 - m1)
    z = 1.0 + e
    pack = jnp.concatenate([a1, a2, 1.0 / z, e / z], axis=0).T
    idx_ref[...] = pack[:, 0:2].astype(jnp.int32)
    wout_ref[...] = pack[:, 2:4]


@functools.partial(jax.jit, static_argnames=("block_t",))
def _run(hidden_states, feature_selectors, thresholds, log_temperatures,
         w_leaf, b_leaf, block_t=512):
    b, l, h = hidden_states.shape
    n = b * l
    x = hidden_states.reshape(n, h)
    num_experts = w_leaf.shape[0]

    sel = pl.pallas_call(
        _entmax_prep_body,
        out_shape=jax.ShapeDtypeStruct(feature_selectors.shape, jnp.float32),
    )(feature_selectors)

    selt = jnp.pad(sel.T, ((0, 0), (0, 8 - _DEPTH)))
    thr_col = jnp.pad(thresholds, ((0, 8 - _DEPTH), (0, 0)))
    lt_col = jnp.pad(log_temperatures, ((0, 8 - _DEPTH), (0, 0)))
    b_col = b_leaf.reshape(num_experts, 1)

    grid = (n // block_t,)
    logits, idx, w = pl.pallas_call(
        _router_body,
        grid=grid,
        in_specs=[
            pl.BlockSpec((block_t, h), lambda i: (i, 0)),
            pl.BlockSpec((h, 8), lambda i: (0, 0)),
            pl.BlockSpec((8, 1), lambda i: (0, 0)),
            pl.BlockSpec((8, 1), lambda i: (0, 0)),
            pl.BlockSpec((num_experts, _NUM_LEAVES), lambda i: (0, 0)),
            pl.BlockSpec((num_experts, 1), lambda i: (0, 0)),
        ],
        out_specs=[
            pl.BlockSpec((block_t, num_experts), lambda i: (i, 0)),
            pl.BlockSpec((block_t, _TOP_K), lambda i: (i, 0)),
            pl.BlockSpec((block_t, _TOP_K), lambda i: (i, 0)),
        ],
        out_shape=[
            jax.ShapeDtypeStruct((n, num_experts), jnp.float32),
            jax.ShapeDtypeStruct((n, _TOP_K), jnp.int32),
            jax.ShapeDtypeStruct((n, _TOP_K), jnp.float32),
        ],
        compiler_params=pltpu.CompilerParams(
            dimension_semantics=("arbitrary",),
        ),
    )(x, selt, thr_col, lt_col, w_leaf, b_col)

    return (logits.reshape(b, l, num_experts),
            idx.reshape(b, l, _TOP_K),
            w.reshape(b, l, _TOP_K))


def kernel(hidden_states, feature_selectors, thresholds, log_temperatures,
           w_leaf, b_leaf):
    return _run(hidden_states, feature_selectors, thresholds,
                log_temperatures, w_leaf, b_leaf)


# speedup vs baseline: 1.3313x; 1.3313x over previous
import functools

import jax
import jax.numpy as jnp
from jax.experimental import pallas as pl
from jax.experimental.pallas import tpu as pltpu

_ALPHA = 1.5
_DEPTH = 6
_NUM_LEAVES = 64
_TOP_K = 2
_N_ITER = 30


def _entmax_prep_body(fs_ref, sel_ref):
    x = fs_ref[...] * (_ALPHA - 1.0)
    max_val = jnp.max(x, axis=-1, keepdims=True)
    tau_lo = max_val - 1.0
    tau_hi = max_val

    def p_fn(tau):
        c = jnp.maximum(x - tau, 0.0)
        return c * c

    f_lo = jnp.sum(p_fn(tau_lo), axis=-1, keepdims=True) - 1.0
    for _ in range(_N_ITER):
        tau_m = 0.5 * (tau_lo + tau_hi)
        f_m = jnp.sum(p_fn(tau_m), axis=-1, keepdims=True) - 1.0
        same_sign = (f_m * f_lo) >= 0.0
        tau_lo = jnp.where(same_sign, tau_m, tau_lo)
        f_lo = jnp.where(same_sign, f_m, f_lo)
        tau_hi = jnp.where(same_sign, tau_hi, tau_m)
    p = p_fn(0.5 * (tau_lo + tau_hi))
    sel_ref[...] = p / jnp.sum(p, axis=-1, keepdims=True)


def _router_body(x_ref, selt_ref, thr_ref, lt_ref, w_ref_, b_ref,
                 logits_ref, idx_ref, wout_ref):
    t = x_ref.shape[0]
    fc = jnp.dot(x_ref[...], selt_ref[...], preferred_element_type=jnp.float32)
    fct = fc.T
    temp = jnp.exp(lt_ref[...])
    scaled = (fct - thr_ref[...]) / temp
    u = scaled * (_ALPHA - 1.0)
    max_val = jnp.maximum(u, 0.0)
    tau_lo = max_val - 1.0
    tau_hi = max_val

    def p2_fn(tau):
        c0 = jnp.maximum(u - tau, 0.0)
        c1 = jnp.maximum(-tau, 0.0)
        return c0 * c0, c1 * c1

    p0, p1 = p2_fn(tau_lo)
    f_lo = (p0 + p1) - 1.0
    for _ in range(_N_ITER):
        tau_m = 0.5 * (tau_lo + tau_hi)
        p0, p1 = p2_fn(tau_m)
        f_m = (p0 + p1) - 1.0
        same_sign = (f_m * f_lo) >= 0.0
        tau_lo = jnp.where(same_sign, tau_m, tau_lo)
        f_lo = jnp.where(same_sign, f_m, f_lo)
        tau_hi = jnp.where(same_sign, tau_hi, tau_m)
    p0, p1 = p2_fn(0.5 * (tau_lo + tau_hi))
    right = p0 / (p0 + p1)
    left = 1.0 - right
    bits_i = jax.lax.broadcasted_iota(jnp.int32, (_NUM_LEAVES, 1), 0)
    leaf = jnp.ones((_NUM_LEAVES, t), dtype=jnp.float32)
    for i in range(_DEPTH):
        bit_col = ((bits_i >> (_DEPTH - 1 - i)) & 1) == 1
        r_i = right[i:i + 1, :]
        l_i = left[i:i + 1, :]
        leaf = leaf * jnp.where(bit_col, r_i, l_i)
    logits = jnp.dot(w_ref_[...], leaf, preferred_element_type=jnp.float32)
    logits = logits + b_ref[...]
    logits_ref[...] = logits.T
    iota_f = jax.lax.broadcasted_iota(jnp.int32, (_NUM_LEAVES, t), 0).astype(
        jnp.float32)
    m1 = jnp.max(logits, axis=0, keepdims=True)
    a1 = jnp.min(jnp.where(logits == m1, iota_f, 64.0), axis=0, keepdims=True)
    masked = jnp.where(iota_f == a1, -jnp.inf, logits)
    m2 = jnp.max(masked, axis=0, keepdims=True)
    a2 = jnp.min(jnp.where(masked == m2, iota_f, 64.0), axis=0, keepdims=True)
    e = jnp.exp(m2 - m1)
    z = 1.0 + e
    pack = jnp.concatenate([a1, a2, 1.0 / z, e / z], axis=0).T
    idx_ref[...] = pack[:, 0:2].astype(jnp.int32)
    wout_ref[...] = pack[:, 2:4]


@functools.partial(jax.jit, static_argnames=("block_t",))
def _run(hidden_states, feature_selectors, thresholds, log_temperatures,
         w_leaf, b_leaf, block_t=2048):
    b, l, h = hidden_states.shape
    n = b * l
    x = hidden_states.reshape(n, h)
    num_experts = w_leaf.shape[0]

    sel = pl.pallas_call(
        _entmax_prep_body,
        out_shape=jax.ShapeDtypeStruct(feature_selectors.shape, jnp.float32),
    )(feature_selectors)

    selt = jnp.pad(sel.T, ((0, 0), (0, 8 - _DEPTH)))
    thr_col = jnp.pad(thresholds, ((0, 8 - _DEPTH), (0, 0)))
    lt_col = jnp.pad(log_temperatures, ((0, 8 - _DEPTH), (0, 0)))
    b_col = b_leaf.reshape(num_experts, 1)

    grid = (n // block_t,)
    logits, idx, w = pl.pallas_call(
        _router_body,
        grid=grid,
        in_specs=[
            pl.BlockSpec((block_t, h), lambda i: (i, 0)),
            pl.BlockSpec((h, 8), lambda i: (0, 0)),
            pl.BlockSpec((8, 1), lambda i: (0, 0)),
            pl.BlockSpec((8, 1), lambda i: (0, 0)),
            pl.BlockSpec((num_experts, _NUM_LEAVES), lambda i: (0, 0)),
            pl.BlockSpec((num_experts, 1), lambda i: (0, 0)),
        ],
        out_specs=[
            pl.BlockSpec((block_t, num_experts), lambda i: (i, 0)),
            pl.BlockSpec((block_t, _TOP_K), lambda i: (i, 0)),
            pl.BlockSpec((block_t, _TOP_K), lambda i: (i, 0)),
        ],
        out_shape=[
            jax.ShapeDtypeStruct((n, num_experts), jnp.float32),
            jax.ShapeDtypeStruct((n, _TOP_K), jnp.int32),
            jax.ShapeDtypeStruct((n, _TOP_K), jnp.float32),
        ],
        compiler_params=pltpu.CompilerParams(
            dimension_semantics=("arbitrary",),
        ),
    )(x, selt, thr_col, lt_col, w_leaf, b_col)

    return (logits.reshape(b, l, num_experts),
            idx.reshape(b, l, _TOP_K),
            w.reshape(b, l, _TOP_K))


def kernel(hidden_states, feature_selectors, thresholds, log_temperatures,
           w_leaf, b_leaf):
    return _run(hidden_states, feature_selectors, thresholds,
                log_temperatures, w_leaf, b_leaf)


# speedup vs baseline: 1.3881x; 1.0426x over previous
import functools

import jax
import jax.numpy as jnp
from jax.experimental import pallas as pl
from jax.experimental.pallas import tpu as pltpu

_ALPHA = 1.5
_DEPTH = 6
_NUM_LEAVES = 64
_TOP_K = 2
_N_ITER = 30


def _entmax_prep_body(fs_ref, sel_ref):
    x = fs_ref[...] * (_ALPHA - 1.0)
    max_val = jnp.max(x, axis=-1, keepdims=True)
    tau_lo = max_val - 1.0
    tau_hi = max_val

    def p_fn(tau):
        c = jnp.maximum(x - tau, 0.0)
        return c * c

    f_lo = jnp.sum(p_fn(tau_lo), axis=-1, keepdims=True) - 1.0
    for _ in range(_N_ITER):
        tau_m = 0.5 * (tau_lo + tau_hi)
        f_m = jnp.sum(p_fn(tau_m), axis=-1, keepdims=True) - 1.0
        same_sign = (f_m * f_lo) >= 0.0
        tau_lo = jnp.where(same_sign, tau_m, tau_lo)
        f_lo = jnp.where(same_sign, f_m, f_lo)
        tau_hi = jnp.where(same_sign, tau_hi, tau_m)
    p = p_fn(0.5 * (tau_lo + tau_hi))
    sel_ref[...] = p / jnp.sum(p, axis=-1, keepdims=True)


def _router_body(x_ref, selt_ref, thr_ref, lt_ref, w_ref_, b_ref,
                 logits_ref, idx_ref, wout_ref):
    t = x_ref.shape[0]
    fc = jnp.dot(x_ref[...], selt_ref[...], preferred_element_type=jnp.float32)
    fct = fc.T
    temp = jnp.exp(lt_ref[...])
    scaled = (fct - thr_ref[...]) / temp
    u = scaled * (_ALPHA - 1.0)
    max_val = jnp.maximum(u, 0.0)
    tau_lo = max_val - 1.0
    tau_hi = max_val

    def p2_fn(tau):
        c0 = jnp.maximum(u - tau, 0.0)
        c1 = jnp.maximum(-tau, 0.0)
        return c0 * c0, c1 * c1

    p0, p1 = p2_fn(tau_lo)
    f_lo = (p0 + p1) - 1.0
    for _ in range(_N_ITER):
        tau_m = 0.5 * (tau_lo + tau_hi)
        p0, p1 = p2_fn(tau_m)
        f_m = (p0 + p1) - 1.0
        same_sign = (f_m * f_lo) >= 0.0
        tau_lo = jnp.where(same_sign, tau_m, tau_lo)
        f_lo = jnp.where(same_sign, f_m, f_lo)
        tau_hi = jnp.where(same_sign, tau_hi, tau_m)
    p0, p1 = p2_fn(0.5 * (tau_lo + tau_hi))
    right = p0 / (p0 + p1)
    left = 1.0 - right
    bits_i = jax.lax.broadcasted_iota(jnp.int32, (_NUM_LEAVES, 1), 0)
    leaf = jnp.ones((_NUM_LEAVES, t), dtype=jnp.float32)
    for i in range(_DEPTH):
        bit_col = ((bits_i >> (_DEPTH - 1 - i)) & 1) == 1
        r_i = right[i:i + 1, :]
        l_i = left[i:i + 1, :]
        leaf = leaf * jnp.where(bit_col, r_i, l_i)
    logits = jnp.dot(w_ref_[...], leaf, preferred_element_type=jnp.float32)
    logits = logits + b_ref[...]
    logits_ref[...] = logits.T
    iota_f = jax.lax.broadcasted_iota(jnp.int32, (_NUM_LEAVES, t), 0).astype(
        jnp.float32)
    m1 = jnp.max(logits, axis=0, keepdims=True)
    a1 = jnp.min(jnp.where(logits == m1, iota_f, 64.0), axis=0, keepdims=True)
    masked = jnp.where(iota_f == a1, -jnp.inf, logits)
    m2 = jnp.max(masked, axis=0, keepdims=True)
    a2 = jnp.min(jnp.where(masked == m2, iota_f, 64.0), axis=0, keepdims=True)
    e = jnp.exp(m2 - m1)
    z = 1.0 + e
    pack = jnp.concatenate([a1, a2, 1.0 / z, e / z], axis=0).T
    idx_ref[...] = pack[:, 0:2].astype(jnp.int32)
    wout_ref[...] = pack[:, 2:4]


@functools.partial(jax.jit, static_argnames=("block_t",))
def _run(hidden_states, feature_selectors, thresholds, log_temperatures,
         w_leaf, b_leaf, block_t=4096):
    b, l, h = hidden_states.shape
    n = b * l
    x = hidden_states.reshape(n, h)
    num_experts = w_leaf.shape[0]

    sel = pl.pallas_call(
        _entmax_prep_body,
        out_shape=jax.ShapeDtypeStruct(feature_selectors.shape, jnp.float32),
    )(feature_selectors)

    selt = jnp.pad(sel.T, ((0, 0), (0, 8 - _DEPTH)))
    thr_col = jnp.pad(thresholds, ((0, 8 - _DEPTH), (0, 0)))
    lt_col = jnp.pad(log_temperatures, ((0, 8 - _DEPTH), (0, 0)))
    b_col = b_leaf.reshape(num_experts, 1)

    grid = (n // block_t,)
    logits, idx, w = pl.pallas_call(
        _router_body,
        grid=grid,
        in_specs=[
            pl.BlockSpec((block_t, h), lambda i: (i, 0)),
            pl.BlockSpec((h, 8), lambda i: (0, 0)),
            pl.BlockSpec((8, 1), lambda i: (0, 0)),
            pl.BlockSpec((8, 1), lambda i: (0, 0)),
            pl.BlockSpec((num_experts, _NUM_LEAVES), lambda i: (0, 0)),
            pl.BlockSpec((num_experts, 1), lambda i: (0, 0)),
        ],
        out_specs=[
            pl.BlockSpec((block_t, num_experts), lambda i: (i, 0)),
            pl.BlockSpec((block_t, _TOP_K), lambda i: (i, 0)),
            pl.BlockSpec((block_t, _TOP_K), lambda i: (i, 0)),
        ],
        out_shape=[
            jax.ShapeDtypeStruct((n, num_experts), jnp.float32),
            jax.ShapeDtypeStruct((n, _TOP_K), jnp.int32),
            jax.ShapeDtypeStruct((n, _TOP_K), jnp.float32),
        ],
        compiler_params=pltpu.CompilerParams(
            dimension_semantics=("arbitrary",),
        ),
    )(x, selt, thr_col, lt_col, w_leaf, b_col)

    return (logits.reshape(b, l, num_experts),
            idx.reshape(b, l, _TOP_K),
            w.reshape(b, l, _TOP_K))


def kernel(hidden_states, feature_selectors, thresholds, log_temperatures,
           w_leaf, b_leaf):
    return _run(hidden_states, feature_selectors, thresholds,
                log_temperatures, w_leaf, b_leaf)


# speedup vs baseline: 2.6908x; 1.9386x over previous
import functools

import jax
import jax.numpy as jnp
from jax.experimental import pallas as pl
from jax.experimental.pallas import tpu as pltpu

_ALPHA = 1.5
_DEPTH = 6
_NUM_LEAVES = 64
_TOP_K = 2
_N_ITER = 30


def _entmax_prep_body(fs_ref, sel_ref):
    x = fs_ref[...] * (_ALPHA - 1.0)
    max_val = jnp.max(x, axis=-1, keepdims=True)
    tau_lo = max_val - 1.0
    tau_hi = max_val

    def p_fn(tau):
        c = jnp.maximum(x - tau, 0.0)
        return c * c

    f_lo = jnp.sum(p_fn(tau_lo), axis=-1, keepdims=True) - 1.0
    for _ in range(_N_ITER):
        tau_m = 0.5 * (tau_lo + tau_hi)
        f_m = jnp.sum(p_fn(tau_m), axis=-1, keepdims=True) - 1.0
        same_sign = (f_m * f_lo) >= 0.0
        tau_lo = jnp.where(same_sign, tau_m, tau_lo)
        f_lo = jnp.where(same_sign, f_m, f_lo)
        tau_hi = jnp.where(same_sign, tau_hi, tau_m)
    p = p_fn(0.5 * (tau_lo + tau_hi))
    sel_ref[...] = p / jnp.sum(p, axis=-1, keepdims=True)


def _router_body(x_ref, selt_ref, thr_ref, lt_ref, w_ref_, b_ref,
                 logits_ref, idx_ref, wout_ref):
    t = x_ref.shape[1]
    fc = jnp.dot(x_ref[0], selt_ref[...], preferred_element_type=jnp.float32)
    fct = fc.T
    temp = jnp.exp(lt_ref[...])
    scaled = (fct - thr_ref[...]) / temp
    u = scaled * (_ALPHA - 1.0)
    max_val = jnp.maximum(u, 0.0)
    tau_lo = max_val - 1.0
    tau_hi = max_val

    def p2_fn(tau):
        c0 = jnp.maximum(u - tau, 0.0)
        c1 = jnp.maximum(-tau, 0.0)
        return c0 * c0, c1 * c1

    p0, p1 = p2_fn(tau_lo)
    f_lo = (p0 + p1) - 1.0
    for _ in range(_N_ITER):
        tau_m = 0.5 * (tau_lo + tau_hi)
        p0, p1 = p2_fn(tau_m)
        f_m = (p0 + p1) - 1.0
        same_sign = (f_m * f_lo) >= 0.0
        tau_lo = jnp.where(same_sign, tau_m, tau_lo)
        f_lo = jnp.where(same_sign, f_m, f_lo)
        tau_hi = jnp.where(same_sign, tau_hi, tau_m)
    p0, p1 = p2_fn(0.5 * (tau_lo + tau_hi))
    right = p0 / (p0 + p1)
    left = 1.0 - right
    bits_i = jax.lax.broadcasted_iota(jnp.int32, (_NUM_LEAVES, 1), 0)
    leaf = jnp.ones((_NUM_LEAVES, t), dtype=jnp.float32)
    for i in range(_DEPTH):
        bit_col = ((bits_i >> (_DEPTH - 1 - i)) & 1) == 1
        r_i = right[i:i + 1, :]
        l_i = left[i:i + 1, :]
        leaf = leaf * jnp.where(bit_col, r_i, l_i)
    logits = jnp.dot(w_ref_[...], leaf, preferred_element_type=jnp.float32)
    logits = logits + b_ref[...]
    logits_ref[...] = logits[None]
    iota_f = jax.lax.broadcasted_iota(jnp.int32, (_NUM_LEAVES, t), 0).astype(
        jnp.float32)
    m1 = jnp.max(logits, axis=0, keepdims=True)
    a1 = jnp.min(jnp.where(logits == m1, iota_f, 64.0), axis=0, keepdims=True)
    masked = jnp.where(iota_f == a1, -jnp.inf, logits)
    m2 = jnp.max(masked, axis=0, keepdims=True)
    a2 = jnp.min(jnp.where(masked == m2, iota_f, 64.0), axis=0, keepdims=True)
    e = jnp.exp(m2 - m1)
    z = 1.0 + e
    idx_ref[...] = jnp.concatenate([a1, a2], axis=0).astype(jnp.int32)[None]
    wout_ref[...] = jnp.concatenate([1.0 / z, e / z], axis=0)[None]


@functools.partial(jax.jit, static_argnames=("block_t",))
def _run(hidden_states, feature_selectors, thresholds, log_temperatures,
         w_leaf, b_leaf, block_t=4096):
    b, l, h = hidden_states.shape
    num_experts = w_leaf.shape[0]

    sel = pl.pallas_call(
        _entmax_prep_body,
        out_shape=jax.ShapeDtypeStruct(feature_selectors.shape, jnp.float32),
    )(feature_selectors)

    selt = jnp.pad(sel.T, ((0, 0), (0, 8 - _DEPTH)))
    thr_col = jnp.pad(thresholds, ((0, 8 - _DEPTH), (0, 0)))
    lt_col = jnp.pad(log_temperatures, ((0, 8 - _DEPTH), (0, 0)))
    b_col = b_leaf.reshape(num_experts, 1)

    grid = (b, l // block_t)
    logits, idx, w = pl.pallas_call(
        _router_body,
        grid=grid,
        in_specs=[
            pl.BlockSpec((1, block_t, h), lambda i, j: (i, j, 0)),
            pl.BlockSpec((h, 8), lambda i, j: (0, 0)),
            pl.BlockSpec((8, 1), lambda i, j: (0, 0)),
            pl.BlockSpec((8, 1), lambda i, j: (0, 0)),
            pl.BlockSpec((num_experts, _NUM_LEAVES), lambda i, j: (0, 0)),
            pl.BlockSpec((num_experts, 1), lambda i, j: (0, 0)),
        ],
        out_specs=[
            pl.BlockSpec((1, num_experts, block_t), lambda i, j: (i, 0, j)),
            pl.BlockSpec((1, _TOP_K, block_t), lambda i, j: (i, 0, j)),
            pl.BlockSpec((1, _TOP_K, block_t), lambda i, j: (i, 0, j)),
        ],
        out_shape=[
            jax.ShapeDtypeStruct((b, num_experts, l), jnp.float32),
            jax.ShapeDtypeStruct((b, _TOP_K, l), jnp.int32),
            jax.ShapeDtypeStruct((b, _TOP_K, l), jnp.float32),
        ],
        compiler_params=pltpu.CompilerParams(
            dimension_semantics=("arbitrary", "arbitrary"),
        ),
    )(hidden_states, selt, thr_col, lt_col, w_leaf, b_col)

    return (jnp.transpose(logits, (0, 2, 1)),
            jnp.transpose(idx, (0, 2, 1)),
            jnp.transpose(w, (0, 2, 1)))


def kernel(hidden_states, feature_selectors, thresholds, log_temperatures,
           w_leaf, b_leaf):
    return _run(hidden_states, feature_selectors, thresholds,
                log_temperatures, w_leaf, b_leaf)
